# Initial kernel scaffold; baseline (speedup 1.0000x reference)
import jax
import jax.numpy as jnp
from jax.experimental import pallas as pl


def kernel(source_nodes, destination_nodes, negative_nodes, edge_times, edge_idxs, neighbors, neighbor_edge_idxs, neighbor_times, memory, edge_raw_features, time_w, time_b, msg_w1, msg_b1, msg_w2, msg_b2, gru_wih, gru_whh, gru_bih, gru_bhh, wq, wk, wv, wo, fc1_w, fc1_b, fc2_w, fc2_b):
    raise NotImplementedError("write your pallas kernel here")


# speedup vs baseline: 3.2796x; 3.2796x over previous
import functools

import jax
import jax.numpy as jnp
from jax import lax
from jax.experimental import pallas as pl
from jax.experimental.pallas import tpu as pltpu
from jax.experimental.pallas import tpu_sc as plsc

NC = 2
NS = 16
NW = NC * NS


def _wid():
    return lax.axis_index("s") * NC + lax.axis_index("c")


def _k1_body(mem_hbm, erf_hbm, pos2_hbm, eidx2_hbm, mempos_out, efeat_out,
             idx_v, rows_v, erows_v, sem):
    w = _wid()
    for j in range(2):
        pltpu.sync_copy(pos2_hbm.at[w * 2 + j], idx_v)
        pltpu.async_copy(mem_hbm.at[idx_v], rows_v, sem).wait()
        pltpu.sync_copy(rows_v, mempos_out.at[pl.ds((w * 2 + j) * 128, 128)])
    pltpu.sync_copy(eidx2_hbm.at[w], idx_v)
    pltpu.async_copy(erf_hbm.at[idx_v], erows_v, sem).wait()
    pltpu.sync_copy(erows_v, efeat_out.at[pl.ds(w * 128, 128)])


def _k2_body(a_ref, b_ref, ef_ref, et_ref, tw_ref, tb_ref,
             w1a_ref, w1b_ref, w1e_ref, w1t_ref, b1_ref, w2_ref, b2_ref,
             out_ref):
    a = a_ref[...]
    b = b_ref[...]
    ef = ef_ref[...]
    et = et_ref[...]
    tw = tw_ref[0:1, :]
    tb = tb_ref[0:1, :]
    tenc = jnp.cos(et * tw + tb)
    h = (jnp.dot(a, w1a_ref[...], preferred_element_type=jnp.float32)
         + jnp.dot(b, w1b_ref[...], preferred_element_type=jnp.float32)
         + jnp.dot(ef, w1e_ref[...], preferred_element_type=jnp.float32)
         + jnp.dot(tenc, w1t_ref[...], preferred_element_type=jnp.float32)
         + b1_ref[0:1, :])
    h = jnp.maximum(h, 0.0)
    out_ref[...] = (jnp.dot(h, w2_ref[...], preferred_element_type=jnp.float32)
                    + b2_ref[0:1, :])


def _k2b_body(pos_ref, out_ref, *, dump):
    p = pos_ref[...]
    eq = p[:, :, None] == p[:, None, :]
    i_idx = lax.broadcasted_iota(jnp.int32, (512, 16, 16), 1)
    k_idx = lax.broadcasted_iota(jnp.int32, (512, 16, 16), 2)
    dup_after = jnp.any(eq & (k_idx > i_idx), axis=2)
    out_ref[...] = jnp.where(dup_after, dump, p)


def _k3_body(sidx_hbm, pos_hbm, msg_hbm, msgwin_out,
             tag_v, sidx_v, posl_v, win_v, mrows_v, sem):
    w = _wid()
    pltpu.sync_copy(sidx_hbm, sidx_v)
    pltpu.sync_copy(pos_hbm.at[pl.ds(w * 256, 256)], posl_v)

    def build(g, c):
        idxv = sidx_v[pl.ds(g * 16, 16)]
        payload = g * 16 + lax.iota(jnp.int32, 16)
        plsc.store_scatter(tag_v, [idxv], payload)
        return c

    lax.fori_loop(0, 512, build, 0)

    for j in range(2):
        for t in range(8):
            pv = posl_v[pl.ds(j * 128 + t * 16, 16)]
            win_v[pl.ds(t * 16, 16)] = plsc.load_gather(tag_v, [pv])
        pltpu.async_copy(msg_hbm.at[win_v], mrows_v, sem).wait()
        pltpu.sync_copy(mrows_v,
                        msgwin_out.at[pl.ds(w * 256 + j * 128, 128)])


def _k4_body(x_ref, h_ref, wih_ref, whh_ref, bih_ref, bhh_ref, out_ref):
    x = x_ref[...]
    h = h_ref[...]
    gi = jnp.dot(x, wih_ref[...], preferred_element_type=jnp.float32) + bih_ref[0:1, :]
    gh = jnp.dot(h, whh_ref[...], preferred_element_type=jnp.float32) + bhh_ref[0:1, :]
    r = jax.nn.sigmoid(gi[:, 0:128] + gh[:, 0:128])
    z = jax.nn.sigmoid(gi[:, 128:256] + gh[:, 128:256])
    n = jnp.tanh(gi[:, 256:384] + r * gh[:, 256:384])
    out_ref[...] = (1.0 - z) * n + z * h


def _k5_body(upd_hbm, pos2_hbm, nm_ref, idx_v, rows_v, sem):
    w = _wid()
    for j in range(2):
        pltpu.sync_copy(pos2_hbm.at[w * 2 + j], idx_v)
        pltpu.sync_copy(upd_hbm.at[pl.ds((w * 2 + j) * 128, 128)], rows_v)
        pltpu.async_copy(rows_v, nm_ref.at[idx_v], sem).wait()


def _k6_body(nm_hbm, nodes2_hbm, nbr2_hbm, neidx2_hbm, erf_hbm,
             srcf_out, nbrf_out, nbref_out, idx_v, rows_v, erows_v, sem):
    w = _wid()
    for t in range(3):
        pltpu.sync_copy(nodes2_hbm.at[w * 3 + t], idx_v)
        pltpu.async_copy(nm_hbm.at[idx_v], rows_v, sem).wait()
        pltpu.sync_copy(rows_v, srcf_out.at[pl.ds((w * 3 + t) * 128, 128)])

    def body(t, c):
        pltpu.sync_copy(nbr2_hbm.at[w * 48 + t], idx_v)
        pltpu.async_copy(nm_hbm.at[idx_v], rows_v, sem).wait()
        pltpu.sync_copy(rows_v, nbrf_out.at[pl.ds((w * 48 + t) * 128, 128)])
        pltpu.sync_copy(neidx2_hbm.at[w * 48 + t], idx_v)
        pltpu.async_copy(erf_hbm.at[idx_v], erows_v, sem).wait()
        pltpu.sync_copy(erows_v, nbref_out.at[pl.ds((w * 48 + t) * 128, 128)])
        return c

    lax.fori_loop(0, 48, body, 0)


def _k7_body(srcf_ref, nbrf_ref, nbref_ref, nt_ref, ts_ref, tw_ref, tb_ref,
             wqs_ref, wqb_ref, wkm_ref, wkt_ref, wke_ref,
             wvm_ref, wvt_ref, wve_ref, wo_ref,
             f1o_ref, f1s_ref, f1b_ref, f2_ref, f2b_ref, out_ref):
    src = srcf_ref[...]
    nbrm = nbrf_ref[...]
    nbe = nbref_ref[...]
    nt = nt_ref[...]
    ts = ts_ref[...]
    tw = tw_ref[0:1, :]
    tb = tb_ref[0:1, :]

    delta3 = ts[:, None, :] - nt[:, :, None]
    tenc = jnp.cos(delta3 * tw[None] + tb[None]).reshape(2048, 128)

    k = (jnp.dot(nbrm, wkm_ref[...], preferred_element_type=jnp.float32)
         + jnp.dot(tenc, wkt_ref[...], preferred_element_type=jnp.float32)
         + jnp.dot(nbe, wke_ref[...], preferred_element_type=jnp.float32))
    v = (jnp.dot(nbrm, wvm_ref[...], preferred_element_type=jnp.float32)
         + jnp.dot(tenc, wvt_ref[...], preferred_element_type=jnp.float32)
         + jnp.dot(nbe, wve_ref[...], preferred_element_type=jnp.float32))
    qrow = jnp.dot(jnp.cos(tb), wqb_ref[...], preferred_element_type=jnp.float32)
    q = jnp.dot(src, wqs_ref[...], preferred_element_type=jnp.float32) + qrow

    qrep = jnp.broadcast_to(q[:, None, :], (128, 16, 128)).reshape(2048, 128)
    prod = qrep * k
    s1 = jnp.sum(prod[:, 0:64], axis=1, keepdims=True).reshape(128, 16) * 0.125
    s2 = jnp.sum(prod[:, 64:128], axis=1, keepdims=True).reshape(128, 16) * 0.125

    m1 = jnp.max(s1, axis=1, keepdims=True)
    e1 = jnp.exp(s1 - m1)
    a1 = e1 / jnp.sum(e1, axis=1, keepdims=True)
    m2 = jnp.max(s2, axis=1, keepdims=True)
    e2 = jnp.exp(s2 - m2)
    a2 = e2 / jnp.sum(e2, axis=1, keepdims=True)

    a1e = jnp.broadcast_to(a1.reshape(2048, 1), (2048, 64))
    a2e = jnp.broadcast_to(a2.reshape(2048, 1), (2048, 64))
    attn = jnp.concatenate([a1e, a2e], axis=1)

    wsum = (attn * v).reshape(128, 16, 128).sum(axis=1)
    out = jnp.dot(wsum, wo_ref[...], preferred_element_type=jnp.float32)

    hmid = (jnp.dot(out, f1o_ref[...], preferred_element_type=jnp.float32)
            + jnp.dot(src, f1s_ref[...], preferred_element_type=jnp.float32)
            + f1b_ref[0:1, :])
    hmid = jnp.maximum(hmid, 0.0)
    out_ref[...] = (jnp.dot(hmid, f2_ref[...], preferred_element_type=jnp.float32)
                    + f2b_ref[0:1, :])


def kernel(source_nodes, destination_nodes, negative_nodes, edge_times,
           edge_idxs, neighbors, neighbor_edge_idxs, neighbor_times, memory,
           edge_raw_features, time_w, time_b, msg_w1, msg_b1, msg_w2, msg_b2,
           gru_wih, gru_whh, gru_bih, gru_bhh, wq, wk, wv, wo, fc1_w, fc1_b,
           fc2_w, fc2_b):
    N, D = memory.shape
    B = source_nodes.shape[0]
    K = neighbors.shape[1]
    M = 3 * B
    P = 2 * B
    MSG_HID = msg_w1.shape[1]
    TAG = ((N + 16) + 15) // 16 * 16
    DUMP = N + 8

    mesh = plsc.VectorSubcoreMesh(core_axis_name="c", subcore_axis_name="s",
                                  num_cores=NC, num_subcores=NS)
    sc_params = pltpu.CompilerParams(use_tc_tiling_on_sc=False,
                                     needs_layout_passes=False)

    pos = jnp.concatenate([source_nodes, destination_nodes])
    nodes = jnp.concatenate([source_nodes, destination_nodes, negative_nodes])
    pos2 = pos.reshape(P // 128, 128)
    eidx2 = edge_idxs.reshape(B // 128, 128)
    nodes2 = nodes.reshape(M // 128, 128)
    nbr2 = neighbors.reshape(M * K // 128, 128)
    neidx2 = neighbor_edge_idxs.reshape(M * K // 128, 128)
    et_col = edge_times.reshape(B, 1)
    ts_col = jnp.concatenate([edge_times, edge_times, edge_times]).reshape(M, 1)

    w1a = msg_w1[0:D]
    w1b = msg_w1[D:2 * D]
    w1e = msg_w1[2 * D:2 * D + 16]
    w1t = msg_w1[2 * D + 16:]
    w2p = jnp.pad(msg_w2, ((0, 0), (0, D - msg_w2.shape[1])))
    b2p = jnp.pad(msg_b2, (0, D - msg_b2.shape[0]))
    wihp = jnp.pad(gru_wih, ((0, D - gru_wih.shape[0]), (0, 0)))

    def row8(x):
        return jnp.broadcast_to(x[None, :], (8, x.shape[0]))

    tw8, tb8 = row8(time_w), row8(time_b)
    b18, b28 = row8(msg_b1), row8(b2p)
    bih8, bhh8 = row8(gru_bih), row8(gru_bhh)
    f1b8, f2b8 = row8(fc1_b), row8(fc2_b)

    wqs, wqb = wq[0:D], wq[D:]
    wkm, wkt, wke = wk[0:D], wk[D:2 * D], wk[2 * D:]
    wvm, wvt, wve = wv[0:D], wv[D:2 * D], wv[2 * D:]
    f1o, f1s = fc1_w[0:D], fc1_w[D:]

    k1 = pl.kernel(
        _k1_body,
        out_type=(jax.ShapeDtypeStruct((P, D), jnp.float32),
                  jax.ShapeDtypeStruct((B, 16), jnp.float32)),
        mesh=mesh,
        scratch_types=[pltpu.VMEM((128,), jnp.int32),
                       pltpu.VMEM((128, D), jnp.float32),
                       pltpu.VMEM((128, 16), jnp.float32),
                       pltpu.SemaphoreType.DMA],
        compiler_params=sc_params,
    )
    mem_pos, efeat = k1(memory, edge_raw_features, pos2, eidx2)

    msg = pl.pallas_call(
        _k2_body,
        grid=(16,),
        in_specs=[
            pl.BlockSpec((512, D), lambda i: (i, 0)),
            pl.BlockSpec((512, D), lambda i: ((i + 8) % 16, 0)),
            pl.BlockSpec((512, 16), lambda i: (i % 8, 0)),
            pl.BlockSpec((512, 1), lambda i: (i % 8, 0)),
            pl.BlockSpec((8, D), lambda i: (0, 0)),
            pl.BlockSpec((8, D), lambda i: (0, 0)),
            pl.BlockSpec((D, MSG_HID), lambda i: (0, 0)),
            pl.BlockSpec((D, MSG_HID), lambda i: (0, 0)),
            pl.BlockSpec((16, MSG_HID), lambda i: (0, 0)),
            pl.BlockSpec((D, MSG_HID), lambda i: (0, 0)),
            pl.BlockSpec((8, MSG_HID), lambda i: (0, 0)),
            pl.BlockSpec((MSG_HID, D), lambda i: (0, 0)),
            pl.BlockSpec((8, D), lambda i: (0, 0)),
        ],
        out_specs=pl.BlockSpec((512, D), lambda i: (i, 0)),
        out_shape=jax.ShapeDtypeStruct((P, D), jnp.float32),
    )(mem_pos, mem_pos, efeat, et_col, tw8, tb8,
      w1a, w1b, w1e, w1t, b18, w2p, b28)

    sidx = pl.pallas_call(
        functools.partial(_k2b_body, dump=DUMP),
        out_shape=jax.ShapeDtypeStruct((P // 16, 16), jnp.int32),
    )(pos.reshape(P // 16, 16)).reshape(P)

    k3 = pl.kernel(
        _k3_body,
        out_type=jax.ShapeDtypeStruct((P, D), jnp.float32),
        mesh=mesh,
        scratch_types=[pltpu.VMEM((TAG,), jnp.int32),
                       pltpu.VMEM((P,), jnp.int32),
                       pltpu.VMEM((256,), jnp.int32),
                       pltpu.VMEM((128,), jnp.int32),
                       pltpu.VMEM((128, D), jnp.float32),
                       pltpu.SemaphoreType.DMA],
        compiler_params=sc_params,
    )
    msg_win = k3(sidx, pos, msg)

    upd = pl.pallas_call(
        _k4_body,
        grid=(16,),
        in_specs=[
            pl.BlockSpec((512, D), lambda i: (i, 0)),
            pl.BlockSpec((512, D), lambda i: (i, 0)),
            pl.BlockSpec((D, 3 * D), lambda i: (0, 0)),
            pl.BlockSpec((D, 3 * D), lambda i: (0, 0)),
            pl.BlockSpec((8, 3 * D), lambda i: (0, 0)),
            pl.BlockSpec((8, 3 * D), lambda i: (0, 0)),
        ],
        out_specs=pl.BlockSpec((512, D), lambda i: (i, 0)),
        out_shape=jax.ShapeDtypeStruct((P, D), jnp.float32),
    )(msg_win, mem_pos, wihp, gru_whh, bih8, bhh8)

    nm_ref = jax.new_ref(memory)
    k5 = pl.kernel(
        _k5_body,
        out_type=(),
        mesh=mesh,
        scratch_types=[pltpu.VMEM((128,), jnp.int32),
                       pltpu.VMEM((128, D), jnp.float32),
                       pltpu.SemaphoreType.DMA],
        compiler_params=sc_params,
    )
    k5(upd, pos2, nm_ref)
    new_mem = jax.freeze(nm_ref)

    k6 = pl.kernel(
        _k6_body,
        out_type=(jax.ShapeDtypeStruct((M, D), jnp.float32),
                  jax.ShapeDtypeStruct((M * K, D), jnp.float32),
                  jax.ShapeDtypeStruct((M * K, 16), jnp.float32)),
        mesh=mesh,
        scratch_types=[pltpu.VMEM((128,), jnp.int32),
                       pltpu.VMEM((128, D), jnp.float32),
                       pltpu.VMEM((128, 16), jnp.float32),
                       pltpu.SemaphoreType.DMA],
        compiler_params=sc_params,
    )
    src_feat, nbr_feat, nbr_ef = k6(new_mem, nodes2, nbr2, neidx2,
                                    edge_raw_features)

    emb = pl.pallas_call(
        _k7_body,
        grid=(M // 128,),
        in_specs=[
            pl.BlockSpec((128, D), lambda i: (i, 0)),
            pl.BlockSpec((2048, D), lambda i: (i, 0)),
            pl.BlockSpec((2048, 16), lambda i: (i, 0)),
            pl.BlockSpec((128, K), lambda i: (i, 0)),
            pl.BlockSpec((128, 1), lambda i: (i, 0)),
            pl.BlockSpec((8, D), lambda i: (0, 0)),
            pl.BlockSpec((8, D), lambda i: (0, 0)),
            pl.BlockSpec((D, D), lambda i: (0, 0)),
            pl.BlockSpec((D, D), lambda i: (0, 0)),
            pl.BlockSpec((D, D), lambda i: (0, 0)),
            pl.BlockSpec((D, D), lambda i: (0, 0)),
            pl.BlockSpec((16, D), lambda i: (0, 0)),
            pl.BlockSpec((D, D), lambda i: (0, 0)),
            pl.BlockSpec((D, D), lambda i: (0, 0)),
            pl.BlockSpec((16, D), lambda i: (0, 0)),
            pl.BlockSpec((D, D), lambda i: (0, 0)),
            pl.BlockSpec((D, D), lambda i: (0, 0)),
            pl.BlockSpec((D, D), lambda i: (0, 0)),
            pl.BlockSpec((8, D), lambda i: (0, 0)),
            pl.BlockSpec((D, D), lambda i: (0, 0)),
            pl.BlockSpec((8, D), lambda i: (0, 0)),
        ],
        out_specs=pl.BlockSpec((128, D), lambda i: (i, 0)),
        out_shape=jax.ShapeDtypeStruct((M, D), jnp.float32),
    )(src_feat, nbr_feat, nbr_ef, neighbor_times, ts_col, tw8, tb8,
      wqs, wqb, wkm, wkt, wke, wvm, wvt, wve, wo, f1o, f1s, f1b8, fc2_w, f2b8)

    return emb
